# Initial kernel scaffold; baseline (speedup 1.0000x reference)
import jax
import jax.numpy as jnp
from jax.experimental import pallas as pl


def kernel(elem_weights, elem_in_fea, self_fea_idx, nbr_fea_idx, gate_W0, gate_b0, gate_g0, gate_be0, gate_Wout, gate_bout, msg_W0, msg_b0, msg_g0, msg_be0, msg_Wout, msg_bout, pow_param):
    raise NotImplementedError("write your pallas kernel here")


# speedup vs baseline: 1.7996x; 1.7996x over previous
import functools

import jax
import jax.numpy as jnp
from jax import lax
from jax.experimental import pallas as pl
from jax.experimental.pallas import tpu as pltpu
from jax.experimental.pallas import tpu_sc as plsc

N = 10000
M = 320000
D = 128
H = 128
NC = 2
NS = 16
L = 16
NW = NC * NS
NPAD = 10016
NROW = NPAD // NS
ZW = 144
BW = 272

C1 = 80
EPT1 = M // NW
NCHUNK1 = EPT1 // C1

C3 = 48
EPT3 = 10032
MPAD = EPT3 * NW
NCHUNK3 = EPT3 // C3

_mesh = plsc.VectorSubcoreMesh(core_axis_name="c", subcore_axis_name="s")
_sc_params = pltpu.CompilerParams(
    needs_layout_passes=False, use_tc_tiling_on_sc=False)

f32 = jnp.float32
i32 = jnp.int32


@functools.partial(
    pl.kernel,
    out_type=(
        jax.ShapeDtypeStruct((NC, NPAD, D), f32),
        jax.ShapeDtypeStruct((NC, NPAD, 16), f32),
    ),
    mesh=_mesh,
    compiler_params=_sc_params,
    scratch_types=(
        pltpu.VMEM((C1,), i32),
        pltpu.VMEM((C1,), i32),
        pltpu.VMEM((C1, D), f32),
        pltpu.VMEM((C1, 16), f32),
        pltpu.VMEM((C1, 16), f32),
        pltpu.VMEM_SHARED((NPAD, D), f32),
        pltpu.VMEM_SHARED((NPAD, 16), f32),
        pltpu.SemaphoreType.DMA,
    ),
)
def _k1(e_hbm, si_hbm, ni_hbm, zd_hbm, z16_hbm,
        g_out, cnt_out,
        si_v, ni_v, rows_v, ones_s, ones_n, g_sp, cnt_sp, sem):
    cid = lax.axis_index("c")
    sid = lax.axis_index("s")
    w = cid * NS + sid

    io = lax.iota(i32, L)
    oh0 = jnp.where(io == 0, 1.0, 0.0).astype(f32)
    oh1 = jnp.where(io == 1, 1.0, 0.0).astype(f32)

    def init_row(r, _):
        ones_s[r, :] = oh0
        ones_n[r, :] = oh1
        return 0
    lax.fori_loop(0, C1, init_row, 0)

    pltpu.sync_copy(zd_hbm, g_sp.at[pl.ds(sid * NROW, NROW)])
    pltpu.sync_copy(z16_hbm, cnt_sp.at[pl.ds(sid * NROW, NROW)])
    plsc.subcore_barrier()

    def chunk(c, _):
        base = w * EPT1 + c * C1
        pltpu.sync_copy(si_hbm.at[pl.ds(base, C1)], si_v)
        pltpu.sync_copy(ni_hbm.at[pl.ds(base, C1)], ni_v)
        pltpu.async_copy(e_hbm.at[ni_v], rows_v, sem).wait()
        pltpu.sync_copy(rows_v, g_sp.at[si_v], add=True)
        pltpu.sync_copy(ones_s, cnt_sp.at[si_v], add=True)
        pltpu.sync_copy(ones_n, cnt_sp.at[ni_v], add=True)
        return 0
    lax.fori_loop(0, NCHUNK1, chunk, 0)

    plsc.subcore_barrier()
    pltpu.sync_copy(g_sp.at[pl.ds(sid * NROW, NROW)],
                    g_out.at[cid, pl.ds(sid * NROW, NROW)])
    pltpu.sync_copy(cnt_sp.at[pl.ds(sid * NROW, NROW)],
                    cnt_out.at[cid, pl.ds(sid * NROW, NROW)])


def _dot(a, b, ta=False, tb=False):
    dn = (((0 if ta else 1,), (1 if tb else 0,)), ((), ()))
    return lax.dot_general(a, b, dn, precision=lax.Precision.HIGHEST,
                           preferred_element_type=f32)


def _k2a_body(e_ref, gp_ref, cp_ref,
              gw0_ref, gb0_ref, gg0_ref, gbe0_ref,
              mw0_ref, mb0_ref, mg0_ref, mbe0_ref,
              fc_ref):
    e = e_ref[...]
    g = gp_ref[0, :N] + gp_ref[1, :N]
    cnt = cp_ref[0, :N] + cp_ref[1, :N]
    cs = cnt[:, 0:1]
    cn = cnt[:, 1:2]

    ssum = _dot(cs, e, ta=True)
    nsum = _dot(cn, e, ta=True)
    s_ss = _dot(e, cs * e, ta=True)
    s_nn = _dot(e, cn * e, ta=True)
    s_sn = _dot(e, g, ta=True)
    mf = f32(M)
    ones_row = jnp.ones((1, D), f32)

    def fold(w0, b0, g0, be0):
        wa = w0[:, :D]
        wb = w0[:, D:]
        m = (_dot(ssum, wa, tb=True) + _dot(nsum, wb, tb=True)) / mf + b0
        q = (_dot(ones_row, _dot(wa, s_ss) * wa, tb=True)
             + 2.0 * _dot(ones_row, _dot(wa, s_sn) * wb, tb=True)
             + _dot(ones_row, _dot(wb, s_nn) * wb, tb=True))
        eh2 = q / mf + 2.0 * b0 * (m - b0) + b0 * b0
        v = eh2 - m * m
        s = g0 * lax.rsqrt(v + 1e-5)
        t = be0 - m * s
        u = b0 * s + t
        return s, u

    sg, ug = fold(gw0_ref[...], gb0_ref[...], gg0_ref[...], gbe0_ref[...])
    sm_, um = fold(mw0_ref[...], mb0_ref[...], mg0_ref[...], mbe0_ref[...])
    fc_ref[0:1, :] = sg
    fc_ref[1:2, :] = ug
    fc_ref[2:3, :] = sm_
    fc_ref[3:4, :] = um
    fc_ref[4:8, :] = jnp.zeros((4, H), f32)


_k2a = pl.pallas_call(
    _k2a_body,
    out_shape=jax.ShapeDtypeStruct((8, H), f32),
)

BL = 2000


def _k2b_body(e_ref, fc_ref, ew_ref, powp_ref,
              gw0_ref, mw0_ref,
              as_ref, bn_ref):
    e = e_ref[...]
    sg = fc_ref[0:1, :]
    ug = fc_ref[1:2, :]
    sm_ = fc_ref[2:3, :]
    um = fc_ref[3:4, :]
    gwa = gw0_ref[:, :D]
    gwb = gw0_ref[:, D:]
    mwa = mw0_ref[:, :D]
    mwb = mw0_ref[:, D:]
    as_ref[:, :H] = _dot(e, gwa, tb=True) * sg
    as_ref[:, H:] = _dot(e, mwa, tb=True) * sm_
    bn_ref[:, :H] = _dot(e, gwb, tb=True) * sg + ug
    bn_ref[:, H:2 * H] = _dot(e, mwb, tb=True) * sm_ + um
    wpow = ew_ref[...] ** powp_ref[...]
    bn_ref[:, 2 * H:] = jnp.broadcast_to(wpow, (BL, BW - 2 * H))


_k2b = pl.pallas_call(
    _k2b_body,
    grid=(N // BL,),
    in_specs=[
        pl.BlockSpec((BL, D), lambda i: (i, 0)),
        pl.BlockSpec((8, H), lambda i: (0, 0)),
        pl.BlockSpec((BL, 1), lambda i: (i, 0)),
        pl.BlockSpec((1, 1), lambda i: (0, 0)),
        pl.BlockSpec((H, 2 * D), lambda i: (0, 0)),
        pl.BlockSpec((H, 2 * D), lambda i: (0, 0)),
    ],
    out_specs=(
        pl.BlockSpec((BL, 2 * H), lambda i: (i, 0)),
        pl.BlockSpec((BL, BW), lambda i: (i, 0)),
    ),
    out_shape=(
        jax.ShapeDtypeStruct((N, 2 * H), f32),
        jax.ShapeDtypeStruct((N, BW), f32),
    ),
)


@functools.partial(
    pl.kernel,
    out_type=jax.ShapeDtypeStruct((NC, NPAD, ZW), f32),
    mesh=_mesh,
    compiler_params=_sc_params,
    scratch_types=(
        pltpu.VMEM((C3,), i32),
        pltpu.VMEM((C3,), i32),
        pltpu.VMEM((C3, 2 * H), f32),
        pltpu.VMEM((C3, BW), f32),
        pltpu.VMEM((C3, ZW), f32),
        pltpu.VMEM((H,), f32),
        pltpu.VMEM((L,), f32),
        pltpu.VMEM_SHARED((NPAD, ZW), f32),
        pltpu.SemaphoreType.DMA,
        pltpu.SemaphoreType.DMA,
    ),
)
def _k3(as_hbm, bn_hbm, wg_hbm, gb_hbm, si_hbm, ni_hbm, zz_hbm,
        z_out,
        si_v, ni_v, as_v, bn_v, zbuf, wg_v, gb_v, z_sp, sem_a, sem_b):
    cid = lax.axis_index("c")
    sid = lax.axis_index("s")
    w = cid * NS + sid

    pltpu.sync_copy(wg_hbm, wg_v)
    pltpu.sync_copy(gb_hbm, gb_v)

    zero16 = jnp.zeros((L,), f32)

    def zrow(r, _):
        for kk in range(ZW // L):
            zbuf[r, pl.ds(kk * L, L)] = zero16
        return 0
    lax.fori_loop(0, C3, zrow, 0)

    pltpu.sync_copy(zz_hbm, z_sp.at[pl.ds(sid * NROW, NROW)])
    plsc.subcore_barrier()

    io = lax.iota(i32, L)

    def chunk(c, _):
        base = w * EPT3 + c * C3
        pltpu.sync_copy(si_hbm.at[pl.ds(base, C3)], si_v)
        pltpu.sync_copy(ni_hbm.at[pl.ds(base, C3)], ni_v)
        cpa = pltpu.async_copy(as_hbm.at[si_v], as_v, sem_a)
        cpb = pltpu.async_copy(bn_hbm.at[ni_v], bn_v, sem_b)
        cpa.wait()
        cpb.wait()
        gb = gb_v[...]

        def group(g, _):
            rows = io + g * L
            wp16 = plsc.load_gather(bn_v, [rows, jnp.full((L,), 2 * H, i32)])

            def gate_j(jj, acc):
                for kk in range(8):
                    j = jj * 8 + kk
                    col = jnp.full((L,), j, i32)
                    a = plsc.load_gather(as_v, [rows, col])
                    b = plsc.load_gather(bn_v, [rows, col])
                    h = a + b
                    sg = h / (1.0 + jnp.exp(-h))
                    wgj = plsc.load_gather(wg_v, [col])
                    acc = acc + sg * wgj
                return acc
            gacc = lax.fori_loop(0, H // 8, gate_j, jnp.zeros((L,), f32))

            w16 = wp16 * jnp.exp(gacc + gb)
            plsc.store_scatter(zbuf, [rows, jnp.full((L,), H, i32)], w16)

            def msg_j(jj, _):
                for kk in range(8):
                    j = jj * 8 + kk
                    colt = jnp.full((L,), H + j, i32)
                    a = plsc.load_gather(as_v, [rows, colt])
                    b = plsc.load_gather(bn_v, [rows, colt])
                    h = a + b
                    sm = h / (1.0 + jnp.exp(-h))
                    plsc.store_scatter(zbuf, [rows, jnp.full((L,), j, i32)],
                                       w16 * sm)
                return 0
            lax.fori_loop(0, H // 8, msg_j, 0)
            return 0
        lax.fori_loop(0, C3 // L, group, 0)

        pltpu.sync_copy(zbuf, z_sp.at[si_v], add=True)
        return 0
    lax.fori_loop(0, NCHUNK3, chunk, 0)

    plsc.subcore_barrier()
    pltpu.sync_copy(z_sp.at[pl.ds(sid * NROW, NROW)],
                    z_out.at[cid, pl.ds(sid * NROW, NROW)])


def _k4_body(zp_ref, e_ref, mwout_ref, mbout_ref, out_ref):
    z = zp_ref[0, :N, :H] + zp_ref[1, :N, :H]
    den = zp_ref[0, :N, H:H + 1] + zp_ref[1, :N, H:H + 1]
    head = (_dot(z, mwout_ref[...], tb=True) + den * mbout_ref[...])
    head = head / (den + 1e-10)
    out_ref[...] = head + e_ref[...]


_k4 = pl.pallas_call(
    _k4_body,
    out_shape=jax.ShapeDtypeStruct((N, D), f32),
)


def kernel(elem_weights, elem_in_fea, self_fea_idx, nbr_fea_idx,
           gate_W0, gate_b0, gate_g0, gate_be0, gate_Wout, gate_bout,
           msg_W0, msg_b0, msg_g0, msg_be0, msg_Wout, msg_bout, pow_param):
    e = elem_in_fea.astype(f32)
    si = self_fea_idx.astype(i32)
    ni = nbr_fea_idx.astype(i32)

    zd = jnp.zeros((NROW, D), f32)
    z16 = jnp.zeros((NROW, 16), f32)
    zz = jnp.zeros((NROW, ZW), f32)

    g_parts, cnt_parts = _k1(e, si, ni, zd, z16)

    fc = _k2a(
        e, g_parts, cnt_parts,
        gate_W0.astype(f32), gate_b0.reshape(1, H).astype(f32),
        gate_g0.reshape(1, H).astype(f32), gate_be0.reshape(1, H).astype(f32),
        msg_W0.astype(f32), msg_b0.reshape(1, H).astype(f32),
        msg_g0.reshape(1, H).astype(f32), msg_be0.reshape(1, H).astype(f32))

    as_tab, bn_tab = _k2b(
        e, fc, elem_weights.astype(f32),
        pow_param.reshape(1, 1).astype(f32),
        gate_W0.astype(f32), msg_W0.astype(f32))

    wgout = gate_Wout.reshape(H).astype(f32)
    gb16 = jnp.broadcast_to(gate_bout.astype(f32), (L,))

    npad_e = MPAD - M
    si_pad = jnp.concatenate([si, jnp.full((npad_e,), N, i32)])
    ni_pad = jnp.concatenate([ni, jnp.zeros((npad_e,), i32)])

    z_parts = _k3(as_tab, bn_tab, wgout, gb16, si_pad, ni_pad, zz)

    return _k4(z_parts, e, msg_Wout.astype(f32),
               msg_bout.reshape(1, D).astype(f32))


# speedup vs baseline: 2.0214x; 1.1232x over previous
import functools

import jax
import jax.numpy as jnp
from jax import lax
from jax.experimental import pallas as pl
from jax.experimental.pallas import tpu as pltpu
from jax.experimental.pallas import tpu_sc as plsc

N = 10000
M = 320000
D = 128
H = 128
NC = 2
NS = 16
L = 16
NW = NC * NS
NPAD = 10016
NROW = NPAD // NS
ZW = 136
BW = 272

C1 = 80
EPT1 = M // NW
NCHUNK1 = EPT1 // C1

C3 = 32
EPT3 = 10112
MPAD = EPT3 * NW
NCHUNK3 = EPT3 // C3

_mesh = plsc.VectorSubcoreMesh(core_axis_name="c", subcore_axis_name="s")
_sc_params = pltpu.CompilerParams(
    needs_layout_passes=False, use_tc_tiling_on_sc=False)

f32 = jnp.float32
i32 = jnp.int32


@functools.partial(
    pl.kernel,
    out_type=(
        jax.ShapeDtypeStruct((NC, NPAD, D), f32),
        jax.ShapeDtypeStruct((NC, NPAD, 16), f32),
    ),
    mesh=_mesh,
    compiler_params=_sc_params,
    scratch_types=(
        pltpu.VMEM((C1,), i32),
        pltpu.VMEM((C1,), i32),
        pltpu.VMEM((C1, D), f32),
        pltpu.VMEM((C1, 16), f32),
        pltpu.VMEM((C1, 16), f32),
        pltpu.VMEM_SHARED((NPAD, D), f32),
        pltpu.VMEM_SHARED((NPAD, 16), f32),
        pltpu.SemaphoreType.DMA,
    ),
)
def _k1(e_hbm, si_hbm, ni_hbm, zd_hbm, z16_hbm,
        g_out, cnt_out,
        si_v, ni_v, rows_v, ones_s, ones_n, g_sp, cnt_sp, sem):
    cid = lax.axis_index("c")
    sid = lax.axis_index("s")
    w = cid * NS + sid

    io = lax.iota(i32, L)
    oh0 = jnp.where(io == 0, 1.0, 0.0).astype(f32)
    oh1 = jnp.where(io == 1, 1.0, 0.0).astype(f32)

    def init_row(r, _):
        ones_s[r, :] = oh0
        ones_n[r, :] = oh1
        return 0
    lax.fori_loop(0, C1, init_row, 0)

    pltpu.sync_copy(zd_hbm, g_sp.at[pl.ds(sid * NROW, NROW)])
    pltpu.sync_copy(z16_hbm, cnt_sp.at[pl.ds(sid * NROW, NROW)])
    plsc.subcore_barrier()

    def chunk(c, _):
        base = w * EPT1 + c * C1
        pltpu.sync_copy(si_hbm.at[pl.ds(base, C1)], si_v)
        pltpu.sync_copy(ni_hbm.at[pl.ds(base, C1)], ni_v)
        pltpu.async_copy(e_hbm.at[ni_v], rows_v, sem).wait()
        pltpu.sync_copy(rows_v, g_sp.at[si_v], add=True)
        pltpu.sync_copy(ones_s, cnt_sp.at[si_v], add=True)
        pltpu.sync_copy(ones_n, cnt_sp.at[ni_v], add=True)
        return 0
    lax.fori_loop(0, NCHUNK1, chunk, 0)

    plsc.subcore_barrier()
    pltpu.sync_copy(g_sp.at[pl.ds(sid * NROW, NROW)],
                    g_out.at[cid, pl.ds(sid * NROW, NROW)])
    pltpu.sync_copy(cnt_sp.at[pl.ds(sid * NROW, NROW)],
                    cnt_out.at[cid, pl.ds(sid * NROW, NROW)])


def _dot(a, b, ta=False, tb=False):
    dn = (((0 if ta else 1,), (1 if tb else 0,)), ((), ()))
    return lax.dot_general(a, b, dn, precision=lax.Precision.HIGHEST,
                           preferred_element_type=f32)


def _k2a_body(e_ref, gp_ref, cp_ref,
              gw0_ref, gb0_ref, gg0_ref, gbe0_ref,
              mw0_ref, mb0_ref, mg0_ref, mbe0_ref,
              fc_ref):
    e = e_ref[...]
    g = gp_ref[0, :N] + gp_ref[1, :N]
    cnt = cp_ref[0, :N] + cp_ref[1, :N]
    cs = cnt[:, 0:1]
    cn = cnt[:, 1:2]

    ssum = _dot(cs, e, ta=True)
    nsum = _dot(cn, e, ta=True)
    s_ss = _dot(e, cs * e, ta=True)
    s_nn = _dot(e, cn * e, ta=True)
    s_sn = _dot(e, g, ta=True)
    mf = f32(M)
    ones_row = jnp.ones((1, D), f32)

    def fold(w0, b0, g0, be0):
        wa = w0[:, :D]
        wb = w0[:, D:]
        m = (_dot(ssum, wa, tb=True) + _dot(nsum, wb, tb=True)) / mf + b0
        q = (_dot(ones_row, _dot(wa, s_ss) * wa, tb=True)
             + 2.0 * _dot(ones_row, _dot(wa, s_sn) * wb, tb=True)
             + _dot(ones_row, _dot(wb, s_nn) * wb, tb=True))
        eh2 = q / mf + 2.0 * b0 * (m - b0) + b0 * b0
        v = eh2 - m * m
        s = g0 * lax.rsqrt(v + 1e-5)
        t = be0 - m * s
        u = b0 * s + t
        return s, u

    sg, ug = fold(gw0_ref[...], gb0_ref[...], gg0_ref[...], gbe0_ref[...])
    sm_, um = fold(mw0_ref[...], mb0_ref[...], mg0_ref[...], mbe0_ref[...])
    fc_ref[0:1, :] = sg
    fc_ref[1:2, :] = ug
    fc_ref[2:3, :] = sm_
    fc_ref[3:4, :] = um
    fc_ref[4:8, :] = jnp.zeros((4, H), f32)


_k2a = pl.pallas_call(
    _k2a_body,
    out_shape=jax.ShapeDtypeStruct((8, H), f32),
)

BL = 2000


def _k2b_body(e_ref, fc_ref, ew_ref, powp_ref,
              gw0_ref, mw0_ref,
              as_ref, bn_ref):
    e = e_ref[...]
    sg = fc_ref[0:1, :]
    ug = fc_ref[1:2, :]
    sm_ = fc_ref[2:3, :]
    um = fc_ref[3:4, :]
    gwa = gw0_ref[:, :D]
    gwb = gw0_ref[:, D:]
    mwa = mw0_ref[:, :D]
    mwb = mw0_ref[:, D:]
    as_ref[:, :H] = _dot(e, gwa, tb=True) * sg
    as_ref[:, H:] = _dot(e, mwa, tb=True) * sm_
    bn_ref[:, :H] = _dot(e, gwb, tb=True) * sg + ug
    bn_ref[:, H:2 * H] = _dot(e, mwb, tb=True) * sm_ + um
    wpow = ew_ref[...] ** powp_ref[...]
    bn_ref[:, 2 * H:] = jnp.broadcast_to(wpow, (BL, BW - 2 * H))


_k2b = pl.pallas_call(
    _k2b_body,
    grid=(N // BL,),
    in_specs=[
        pl.BlockSpec((BL, D), lambda i: (i, 0)),
        pl.BlockSpec((8, H), lambda i: (0, 0)),
        pl.BlockSpec((BL, 1), lambda i: (i, 0)),
        pl.BlockSpec((1, 1), lambda i: (0, 0)),
        pl.BlockSpec((H, 2 * D), lambda i: (0, 0)),
        pl.BlockSpec((H, 2 * D), lambda i: (0, 0)),
    ],
    out_specs=(
        pl.BlockSpec((BL, 2 * H), lambda i: (i, 0)),
        pl.BlockSpec((BL, BW), lambda i: (i, 0)),
    ),
    out_shape=(
        jax.ShapeDtypeStruct((N, 2 * H), f32),
        jax.ShapeDtypeStruct((N, BW), f32),
    ),
)


@functools.partial(
    pl.kernel,
    out_type=jax.ShapeDtypeStruct((NC, NPAD, ZW), f32),
    mesh=_mesh,
    compiler_params=_sc_params,
    scratch_types=(
        pltpu.VMEM((4, C3), i32),
        pltpu.VMEM((4, C3), i32),
        pltpu.VMEM((2 * C3, 2 * H), f32),
        pltpu.VMEM((2 * C3, BW), f32),
        pltpu.VMEM((2 * C3, ZW), f32),
        pltpu.VMEM((H,), f32),
        pltpu.VMEM((L,), f32),
        pltpu.VMEM_SHARED((NPAD, ZW), f32),
        pltpu.SemaphoreType.DMA,
        pltpu.SemaphoreType.DMA,
        pltpu.SemaphoreType.DMA,
        pltpu.SemaphoreType.DMA,
        pltpu.SemaphoreType.DMA,
        pltpu.SemaphoreType.DMA,
    ),
)
def _k3(as_hbm, bn_hbm, wg_hbm, gb_hbm, si_hbm, ni_hbm, zz_hbm,
        z_out,
        si4, ni4, as_v, bn_v, zbuf, wg_v, gb_v, z_sp,
        ga0, ga1, gbs0, gbs1, sc0, sc1):
    cid = lax.axis_index("c")
    sid = lax.axis_index("s")
    w = cid * NS + sid
    tbase = w * EPT3

    ga = (ga0, ga1)
    gbs = (gbs0, gbs1)
    sc = (sc0, sc1)

    pltpu.sync_copy(wg_hbm, wg_v)
    pltpu.sync_copy(gb_hbm, gb_v)

    zero16 = jnp.zeros((L,), f32)

    def zrow(r, _):
        for kk in range(ZW // L):
            zbuf[r, pl.ds(kk * L, L)] = zero16
        return 0
    lax.fori_loop(0, 2 * C3, zrow, 0)

    pltpu.sync_copy(zz_hbm, z_sp.at[pl.ds(sid * NROW, NROW)])
    plsc.subcore_barrier()

    io = lax.iota(i32, L)
    gb16 = gb_v[...]

    def load_idx(slot, c):
        base = tbase + c * C3
        pltpu.sync_copy(si_hbm.at[pl.ds(base, C3)], si4.at[slot])
        pltpu.sync_copy(ni_hbm.at[pl.ds(base, C3)], ni4.at[slot])

    def fire_gather(pd, slot):
        pltpu.async_copy(as_hbm.at[si4.at[slot]],
                         as_v.at[pl.ds(pd * C3, C3)], ga[pd])
        pltpu.async_copy(bn_hbm.at[ni4.at[slot]],
                         bn_v.at[pl.ds(pd * C3, C3)], gbs[pd])

    def wait_gather(pd):
        pltpu.make_async_copy(as_hbm.at[si4.at[0]],
                              as_v.at[pl.ds(pd * C3, C3)], ga[pd]).wait()
        pltpu.make_async_copy(bn_hbm.at[ni4.at[0]],
                              bn_v.at[pl.ds(pd * C3, C3)], gbs[pd]).wait()

    def fire_scatter(pd, slot):
        pltpu.async_copy(zbuf.at[pl.ds(pd * C3, C3)],
                         z_sp.at[si4.at[slot]], sc[pd], add=True)

    def wait_scatter(pd):
        pltpu.make_async_copy(zbuf.at[pl.ds(pd * C3, C3)],
                              z_sp.at[si4.at[0]], sc[pd]).wait()

    def compute(pd):
        ro = pd * C3
        for g in range(C3 // L):
            rows = io + (ro + g * L)
            wp16 = plsc.load_gather(bn_v, [rows, jnp.full((L,), 2 * H, i32)])

            def gate_j(jj, acc):
                for kk in range(8):
                    j = jj * 8 + kk
                    col = jnp.full((L,), j, i32)
                    a = plsc.load_gather(as_v, [rows, col])
                    b = plsc.load_gather(bn_v, [rows, col])
                    h = a + b
                    sg = h / (1.0 + jnp.exp(-h))
                    wgj = plsc.load_gather(wg_v, [col])
                    acc = acc + sg * wgj
                return acc
            gacc = lax.fori_loop(0, H // 8, gate_j, jnp.zeros((L,), f32))

            w16 = wp16 * jnp.exp(gacc + gb16)
            plsc.store_scatter(zbuf, [rows, jnp.full((L,), H, i32)], w16)

            def msg_j(jj, _):
                for kk in range(8):
                    j = jj * 8 + kk
                    colt = jnp.full((L,), H + j, i32)
                    a = plsc.load_gather(as_v, [rows, colt])
                    b = plsc.load_gather(bn_v, [rows, colt])
                    h = a + b
                    sm = h / (1.0 + jnp.exp(-h))
                    plsc.store_scatter(zbuf, [rows, jnp.full((L,), j, i32)],
                                       w16 * sm)
                return 0
            lax.fori_loop(0, H // 8, msg_j, 0)

    load_idx(0, 0)
    fire_gather(0, 0)
    load_idx(1, 1)

    def quad(t, _):
        for r in range(4):
            c = t * 4 + r
            pd = r % 2

            @pl.when(c >= 2)
            def _():
                wait_scatter(pd)

            @pl.when(c + 1 < NCHUNK3)
            def _():
                fire_gather((r + 1) % 2, (r + 1) % 4)

            wait_gather(pd)
            compute(pd)
            fire_scatter(pd, r)

            @pl.when(c + 2 < NCHUNK3)
            def _():
                load_idx((r + 2) % 4, c + 2)
        return 0
    lax.fori_loop(0, NCHUNK3 // 4, quad, 0)

    wait_scatter(0)
    wait_scatter(1)

    plsc.subcore_barrier()
    pltpu.sync_copy(z_sp.at[pl.ds(sid * NROW, NROW)],
                    z_out.at[cid, pl.ds(sid * NROW, NROW)])


def _k4_body(zp_ref, e_ref, mwout_ref, mbout_ref, out_ref):
    z = zp_ref[0, :N, :H] + zp_ref[1, :N, :H]
    den = zp_ref[0, :N, H:H + 1] + zp_ref[1, :N, H:H + 1]
    head = (_dot(z, mwout_ref[...], tb=True) + den * mbout_ref[...])
    head = head / (den + 1e-10)
    out_ref[...] = head + e_ref[...]


_k4 = pl.pallas_call(
    _k4_body,
    out_shape=jax.ShapeDtypeStruct((N, D), f32),
)


def kernel(elem_weights, elem_in_fea, self_fea_idx, nbr_fea_idx,
           gate_W0, gate_b0, gate_g0, gate_be0, gate_Wout, gate_bout,
           msg_W0, msg_b0, msg_g0, msg_be0, msg_Wout, msg_bout, pow_param):
    e = elem_in_fea.astype(f32)
    si = self_fea_idx.astype(i32)
    ni = nbr_fea_idx.astype(i32)

    zd = jnp.zeros((NROW, D), f32)
    z16 = jnp.zeros((NROW, 16), f32)
    zz = jnp.zeros((NROW, ZW), f32)

    g_parts, cnt_parts = _k1(e, si, ni, zd, z16)

    fc = _k2a(
        e, g_parts, cnt_parts,
        gate_W0.astype(f32), gate_b0.reshape(1, H).astype(f32),
        gate_g0.reshape(1, H).astype(f32), gate_be0.reshape(1, H).astype(f32),
        msg_W0.astype(f32), msg_b0.reshape(1, H).astype(f32),
        msg_g0.reshape(1, H).astype(f32), msg_be0.reshape(1, H).astype(f32))

    as_tab, bn_tab = _k2b(
        e, fc, elem_weights.astype(f32),
        pow_param.reshape(1, 1).astype(f32),
        gate_W0.astype(f32), msg_W0.astype(f32))

    wgout = gate_Wout.reshape(H).astype(f32)
    gb16 = jnp.broadcast_to(gate_bout.astype(f32), (L,))

    npad_e = MPAD - M
    si_pad = jnp.concatenate([si, jnp.full((npad_e,), N, i32)])
    ni_pad = jnp.concatenate([ni, jnp.zeros((npad_e,), i32)])

    z_parts = _k3(as_tab, bn_tab, wgout, gb16, si_pad, ni_pad, zz)

    return _k4(z_parts, e, msg_Wout.astype(f32),
               msg_bout.reshape(1, D).astype(f32))


# speedup vs baseline: 3.6846x; 1.8228x over previous
import functools

import jax
import jax.numpy as jnp
from jax import lax
from jax.experimental import pallas as pl
from jax.experimental.pallas import tpu as pltpu
from jax.experimental.pallas import tpu_sc as plsc

N = 10000
M = 320000
D = 128
H = 128
NC = 2
NS = 16
L = 16
NW = NC * NS
NPAD = 10016
NROW = NPAD // NS
ZW = 136
BW = 272

C1 = 80
EPT1 = M // NW
NCHUNK1 = EPT1 // C1

C3 = 32
EPT3 = 10112
MPAD = EPT3 * NW
NCHUNK3 = EPT3 // C3

_mesh = plsc.VectorSubcoreMesh(core_axis_name="c", subcore_axis_name="s")
_sc_params = pltpu.CompilerParams(
    needs_layout_passes=False, use_tc_tiling_on_sc=False)

f32 = jnp.float32
i32 = jnp.int32


@functools.partial(
    pl.kernel,
    out_type=(
        jax.ShapeDtypeStruct((NC, NPAD, D), f32),
        jax.ShapeDtypeStruct((NC, NPAD, 16), f32),
    ),
    mesh=_mesh,
    compiler_params=_sc_params,
    scratch_types=(
        pltpu.VMEM((C1,), i32),
        pltpu.VMEM((C1,), i32),
        pltpu.VMEM((C1, D), f32),
        pltpu.VMEM((C1, 16), f32),
        pltpu.VMEM((C1, 16), f32),
        pltpu.VMEM_SHARED((NPAD, D), f32),
        pltpu.VMEM_SHARED((NPAD, 16), f32),
        pltpu.SemaphoreType.DMA,
    ),
)
def _k1(e_hbm, si_hbm, ni_hbm, zd_hbm, z16_hbm,
        g_out, cnt_out,
        si_v, ni_v, rows_v, ones_s, ones_n, g_sp, cnt_sp, sem):
    cid = lax.axis_index("c")
    sid = lax.axis_index("s")
    w = cid * NS + sid

    io = lax.iota(i32, L)
    oh0 = jnp.where(io == 0, 1.0, 0.0).astype(f32)
    oh1 = jnp.where(io == 1, 1.0, 0.0).astype(f32)

    def init_row(r, _):
        ones_s[r, :] = oh0
        ones_n[r, :] = oh1
        return 0
    lax.fori_loop(0, C1, init_row, 0)

    pltpu.sync_copy(zd_hbm, g_sp.at[pl.ds(sid * NROW, NROW)])
    pltpu.sync_copy(z16_hbm, cnt_sp.at[pl.ds(sid * NROW, NROW)])
    plsc.subcore_barrier()

    def chunk(c, _):
        base = w * EPT1 + c * C1
        pltpu.sync_copy(si_hbm.at[pl.ds(base, C1)], si_v)
        pltpu.sync_copy(ni_hbm.at[pl.ds(base, C1)], ni_v)
        pltpu.async_copy(e_hbm.at[ni_v], rows_v, sem).wait()
        pltpu.sync_copy(rows_v, g_sp.at[si_v], add=True)
        pltpu.sync_copy(ones_s, cnt_sp.at[si_v], add=True)
        pltpu.sync_copy(ones_n, cnt_sp.at[ni_v], add=True)
        return 0
    lax.fori_loop(0, NCHUNK1, chunk, 0)

    plsc.subcore_barrier()
    pltpu.sync_copy(g_sp.at[pl.ds(sid * NROW, NROW)],
                    g_out.at[cid, pl.ds(sid * NROW, NROW)])
    pltpu.sync_copy(cnt_sp.at[pl.ds(sid * NROW, NROW)],
                    cnt_out.at[cid, pl.ds(sid * NROW, NROW)])


def _dot(a, b, ta=False, tb=False):
    dn = (((0 if ta else 1,), (1 if tb else 0,)), ((), ()))
    return lax.dot_general(a, b, dn, precision=lax.Precision.HIGHEST,
                           preferred_element_type=f32)


def _k2a_body(e_ref, gp_ref, cp_ref,
              gw0_ref, gb0_ref, gg0_ref, gbe0_ref,
              mw0_ref, mb0_ref, mg0_ref, mbe0_ref,
              fc_ref):
    e = e_ref[...]
    g = gp_ref[0, :N] + gp_ref[1, :N]
    cnt = cp_ref[0, :N] + cp_ref[1, :N]
    cs = cnt[:, 0:1]
    cn = cnt[:, 1:2]

    ssum = _dot(cs, e, ta=True)
    nsum = _dot(cn, e, ta=True)
    s_ss = _dot(e, cs * e, ta=True)
    s_nn = _dot(e, cn * e, ta=True)
    s_sn = _dot(e, g, ta=True)
    mf = f32(M)
    ones_row = jnp.ones((1, D), f32)

    def fold(w0, b0, g0, be0):
        wa = w0[:, :D]
        wb = w0[:, D:]
        m = (_dot(ssum, wa, tb=True) + _dot(nsum, wb, tb=True)) / mf + b0
        q = (_dot(ones_row, _dot(wa, s_ss) * wa, tb=True)
             + 2.0 * _dot(ones_row, _dot(wa, s_sn) * wb, tb=True)
             + _dot(ones_row, _dot(wb, s_nn) * wb, tb=True))
        eh2 = q / mf + 2.0 * b0 * (m - b0) + b0 * b0
        v = eh2 - m * m
        s = g0 * lax.rsqrt(v + 1e-5)
        t = be0 - m * s
        u = b0 * s + t
        return s, u

    sg, ug = fold(gw0_ref[...], gb0_ref[...], gg0_ref[...], gbe0_ref[...])
    sm_, um = fold(mw0_ref[...], mb0_ref[...], mg0_ref[...], mbe0_ref[...])
    fc_ref[0:1, :] = sg
    fc_ref[1:2, :] = ug
    fc_ref[2:3, :] = sm_
    fc_ref[3:4, :] = um
    fc_ref[4:8, :] = jnp.zeros((4, H), f32)


_k2a = pl.pallas_call(
    _k2a_body,
    out_shape=jax.ShapeDtypeStruct((8, H), f32),
)

BL = 2000


def _k2b_body(e_ref, fc_ref, ew_ref, powp_ref, gbout_ref,
              gw0_ref, mw0_ref,
              as_ref, bn_ref):
    e = e_ref[...]
    sg = fc_ref[0:1, :]
    ug = fc_ref[1:2, :]
    sm_ = fc_ref[2:3, :]
    um = fc_ref[3:4, :]
    gwa = gw0_ref[:, :D]
    gwb = gw0_ref[:, D:]
    mwa = mw0_ref[:, :D]
    mwb = mw0_ref[:, D:]
    as_ref[:, :H] = _dot(e, gwa, tb=True) * (-sg)
    as_ref[:, H:] = _dot(e, mwa, tb=True) * (-sm_)
    bn_ref[:, :H] = _dot(e, gwb, tb=True) * (-sg) - ug
    bn_ref[:, H:2 * H] = _dot(e, mwb, tb=True) * (-sm_) - um
    wpow = (ew_ref[...] ** powp_ref[...]) * jnp.exp(gbout_ref[...])
    bn_ref[:, 2 * H:] = jnp.broadcast_to(wpow, (BL, BW - 2 * H))


_k2b = pl.pallas_call(
    _k2b_body,
    grid=(N // BL,),
    in_specs=[
        pl.BlockSpec((BL, D), lambda i: (i, 0)),
        pl.BlockSpec((8, H), lambda i: (0, 0)),
        pl.BlockSpec((BL, 1), lambda i: (i, 0)),
        pl.BlockSpec((1, 1), lambda i: (0, 0)),
        pl.BlockSpec((1, 1), lambda i: (0, 0)),
        pl.BlockSpec((H, 2 * D), lambda i: (0, 0)),
        pl.BlockSpec((H, 2 * D), lambda i: (0, 0)),
    ],
    out_specs=(
        pl.BlockSpec((BL, 2 * H), lambda i: (i, 0)),
        pl.BlockSpec((BL, BW), lambda i: (i, 0)),
    ),
    out_shape=(
        jax.ShapeDtypeStruct((N, 2 * H), f32),
        jax.ShapeDtypeStruct((N, BW), f32),
    ),
)


@functools.partial(
    pl.kernel,
    out_type=jax.ShapeDtypeStruct((NC, NPAD, ZW), f32),
    mesh=_mesh,
    compiler_params=_sc_params,
    scratch_types=(
        pltpu.VMEM((4, C3), i32),
        pltpu.VMEM((4, C3), i32),
        pltpu.VMEM((2 * C3, 2 * H), f32),
        pltpu.VMEM((2 * C3, BW), f32),
        pltpu.VMEM((2 * C3, ZW), f32),
        pltpu.VMEM((H,), f32),
        pltpu.VMEM_SHARED((NPAD, ZW), f32),
        pltpu.SemaphoreType.DMA,
        pltpu.SemaphoreType.DMA,
        pltpu.SemaphoreType.DMA,
        pltpu.SemaphoreType.DMA,
        pltpu.SemaphoreType.DMA,
        pltpu.SemaphoreType.DMA,
        pltpu.SemaphoreType.DMA,
        pltpu.SemaphoreType.DMA,
    ),
)
def _k3(as_hbm, bn_hbm, wg_hbm, si_hbm, ni_hbm, zz_hbm,
        z_out,
        si4, ni4, as_v, bn_v, zbuf, wg_v, z_sp,
        ga0, ga1, gbs0, gbs1, sc0, sc1, ix0, ix1):
    cid = lax.axis_index("c")
    sid = lax.axis_index("s")
    w = cid * NS + sid
    tbase = w * EPT3

    ga = (ga0, ga1)
    gbs = (gbs0, gbs1)
    sc = (sc0, sc1)
    ix = (ix0, ix1)

    pltpu.sync_copy(wg_hbm, wg_v)

    zero16 = jnp.zeros((L,), f32)

    def zrow(r, _):
        for kk in range(ZW // L):
            zbuf[r, pl.ds(kk * L, L)] = zero16
        return 0
    lax.fori_loop(0, 2 * C3, zrow, 0)

    pltpu.sync_copy(zz_hbm, z_sp.at[pl.ds(sid * NROW, NROW)])
    plsc.subcore_barrier()

    io = lax.iota(i32, L)

    def load_idx(slot, c):
        base = tbase + c * C3
        pltpu.sync_copy(si_hbm.at[pl.ds(base, C3)], si4.at[slot])
        pltpu.sync_copy(ni_hbm.at[pl.ds(base, C3)], ni4.at[slot])

    def fire_idx(slot, c, pd):
        base = tbase + c * C3
        pltpu.async_copy(si_hbm.at[pl.ds(base, C3)], si4.at[slot], ix[pd])
        pltpu.async_copy(ni_hbm.at[pl.ds(base, C3)], ni4.at[slot], ix[pd])

    def wait_idx(pd):
        pltpu.make_async_copy(si_hbm.at[pl.ds(0, C3)], si4.at[0],
                              ix[pd]).wait()
        pltpu.make_async_copy(ni_hbm.at[pl.ds(0, C3)], ni4.at[0],
                              ix[pd]).wait()

    def fire_gather(pd, slot):
        pltpu.async_copy(as_hbm.at[si4.at[slot]],
                         as_v.at[pl.ds(pd * C3, C3)], ga[pd])
        pltpu.async_copy(bn_hbm.at[ni4.at[slot]],
                         bn_v.at[pl.ds(pd * C3, C3)], gbs[pd])

    def wait_gather(pd):
        pltpu.make_async_copy(as_hbm.at[si4.at[0]],
                              as_v.at[pl.ds(pd * C3, C3)], ga[pd]).wait()
        pltpu.make_async_copy(bn_hbm.at[ni4.at[0]],
                              bn_v.at[pl.ds(pd * C3, C3)], gbs[pd]).wait()

    def fire_scatter(pd, slot):
        pltpu.async_copy(zbuf.at[pl.ds(pd * C3, C3)],
                         z_sp.at[si4.at[slot]], sc[pd], add=True)

    def wait_scatter(pd):
        pltpu.make_async_copy(zbuf.at[pl.ds(pd * C3, C3)],
                              z_sp.at[si4.at[0]], sc[pd]).wait()

    zvec_f = jnp.zeros((L,), f32)
    zvec_i = jnp.zeros((L,), i32)

    def nsilu(arg):
        return arg / (1.0 + jnp.exp(arg))

    def compute(pd):
        ro = pd * C3
        rows0 = io + ro
        rows1 = io + (ro + L)
        col_wp = jnp.full((L,), 2 * H, i32)
        wp0 = plsc.load_gather(bn_v, [rows0, col_wp])
        wp1 = plsc.load_gather(bn_v, [rows1, col_wp])

        @plsc.parallel_loop(
            0, H, 1, unroll=4,
            carry=(zvec_f, zvec_f, zvec_f, zvec_f, zvec_i))
        def gate_j(j, car):
            p0, p1, q0, q1, colv = car
            wgj = plsc.load_gather(wg_v, [colv])
            a0 = plsc.load_gather(as_v, [rows0, colv])
            b0 = plsc.load_gather(bn_v, [rows0, colv])
            s0 = nsilu(a0 + b0)
            a1 = plsc.load_gather(as_v, [rows1, colv])
            b1 = plsc.load_gather(bn_v, [rows1, colv])
            s1 = nsilu(a1 + b1)
            return (p1, p0 + s0 * wgj, q1, q0 + s1 * wgj, colv + 1)
        p0, p1, q0, q1, _ = gate_j

        w0 = wp0 * jnp.exp(p0 + p1)
        w1 = wp1 * jnp.exp(q0 + q1)
        wn0 = 0.0 - w0
        wn1 = 0.0 - w1
        colw = jnp.full((L,), H, i32)
        plsc.store_scatter(zbuf, [rows0, colw], w0)
        plsc.store_scatter(zbuf, [rows1, colw], w1)

        @plsc.parallel_loop(
            0, H, 1, unroll=4,
            carry=(zvec_i, jnp.full((L,), H, i32)))
        def msg_j(j, car):
            colz, colh = car
            a0 = plsc.load_gather(as_v, [rows0, colh])
            b0 = plsc.load_gather(bn_v, [rows0, colh])
            plsc.store_scatter(zbuf, [rows0, colz], wn0 * nsilu(a0 + b0))
            a1 = plsc.load_gather(as_v, [rows1, colh])
            b1 = plsc.load_gather(bn_v, [rows1, colh])
            plsc.store_scatter(zbuf, [rows1, colz], wn1 * nsilu(a1 + b1))
            return (colz + 1, colh + 1)
        del msg_j

    load_idx(0, 0)
    fire_gather(0, 0)
    fire_idx(1, 1, 1)

    def quad(t, _):
        for r in range(4):
            c = t * 4 + r
            pd = r % 2

            @pl.when(c >= 2)
            def _():
                wait_scatter(pd)

            @pl.when(c + 1 < NCHUNK3)
            def _():
                wait_idx((r + 1) % 2)
                fire_gather((r + 1) % 2, (r + 1) % 4)

            @pl.when(c + 2 < NCHUNK3)
            def _():
                fire_idx((r + 2) % 4, c + 2, r % 2)

            wait_gather(pd)
            compute(pd)
            fire_scatter(pd, r)
        return 0
    lax.fori_loop(0, NCHUNK3 // 4, quad, 0)

    wait_scatter(0)
    wait_scatter(1)

    plsc.subcore_barrier()
    pltpu.sync_copy(z_sp.at[pl.ds(sid * NROW, NROW)],
                    z_out.at[cid, pl.ds(sid * NROW, NROW)])


def _k4_body(zp_ref, e_ref, mwout_ref, mbout_ref, out_ref):
    z = zp_ref[0, :N, :H] + zp_ref[1, :N, :H]
    den = zp_ref[0, :N, H:H + 1] + zp_ref[1, :N, H:H + 1]
    head = (_dot(z, mwout_ref[...], tb=True) + den * mbout_ref[...])
    head = head / (den + 1e-10)
    out_ref[...] = head + e_ref[...]


_k4 = pl.pallas_call(
    _k4_body,
    out_shape=jax.ShapeDtypeStruct((N, D), f32),
)


def kernel(elem_weights, elem_in_fea, self_fea_idx, nbr_fea_idx,
           gate_W0, gate_b0, gate_g0, gate_be0, gate_Wout, gate_bout,
           msg_W0, msg_b0, msg_g0, msg_be0, msg_Wout, msg_bout, pow_param):
    e = elem_in_fea.astype(f32)
    si = self_fea_idx.astype(i32)
    ni = nbr_fea_idx.astype(i32)

    zd = jnp.zeros((NROW, D), f32)
    z16 = jnp.zeros((NROW, 16), f32)
    zz = jnp.zeros((NROW, ZW), f32)

    g_parts, cnt_parts = _k1(e, si, ni, zd, z16)

    fc = _k2a(
        e, g_parts, cnt_parts,
        gate_W0.astype(f32), gate_b0.reshape(1, H).astype(f32),
        gate_g0.reshape(1, H).astype(f32), gate_be0.reshape(1, H).astype(f32),
        msg_W0.astype(f32), msg_b0.reshape(1, H).astype(f32),
        msg_g0.reshape(1, H).astype(f32), msg_be0.reshape(1, H).astype(f32))

    as_tab, bn_tab = _k2b(
        e, fc, elem_weights.astype(f32),
        pow_param.reshape(1, 1).astype(f32),
        gate_bout.reshape(1, 1).astype(f32),
        gate_W0.astype(f32), msg_W0.astype(f32))

    wgout = (-gate_Wout).reshape(H).astype(f32)

    npad_e = MPAD - M
    si_pad = jnp.concatenate([si, jnp.full((npad_e,), N, i32)])
    ni_pad = jnp.concatenate([ni, jnp.zeros((npad_e,), i32)])

    z_parts = _k3(as_tab, bn_tab, wgout, si_pad, ni_pad, zz)

    return _k4(z_parts, e, msg_Wout.astype(f32),
               msg_bout.reshape(1, D).astype(f32))


# speedup vs baseline: 3.8336x; 1.0404x over previous
import functools

import jax
import jax.numpy as jnp
from jax import lax
from jax.experimental import pallas as pl
from jax.experimental.pallas import tpu as pltpu
from jax.experimental.pallas import tpu_sc as plsc

N = 10000
M = 320000
D = 128
H = 128
NC = 2
NS = 16
L = 16
NW = NC * NS
NPAD = 10016
NROW = NPAD // NS
ZW = 136
BW = 272

C1 = 80
EPT1 = M // NW
NCHUNK1 = EPT1 // C1

C3 = 32
EPT3 = 10112
MPAD = EPT3 * NW
NCHUNK3 = EPT3 // C3

_mesh = plsc.VectorSubcoreMesh(core_axis_name="c", subcore_axis_name="s")
_sc_params = pltpu.CompilerParams(
    needs_layout_passes=False, use_tc_tiling_on_sc=False)

f32 = jnp.float32
i32 = jnp.int32


@functools.partial(
    pl.kernel,
    out_type=(
        jax.ShapeDtypeStruct((NC, NPAD, D), f32),
        jax.ShapeDtypeStruct((NC, NPAD, 16), f32),
    ),
    mesh=_mesh,
    compiler_params=_sc_params,
    scratch_types=(
        pltpu.VMEM((C1,), i32),
        pltpu.VMEM((C1,), i32),
        pltpu.VMEM((C1, D), f32),
        pltpu.VMEM((C1, 16), f32),
        pltpu.VMEM((C1, 16), f32),
        pltpu.VMEM_SHARED((NPAD, D), f32),
        pltpu.VMEM_SHARED((NPAD, 16), f32),
        pltpu.SemaphoreType.DMA,
    ),
)
def _k1(e_hbm, si_hbm, ni_hbm, zd_hbm, z16_hbm,
        g_out, cnt_out,
        si_v, ni_v, rows_v, ones_s, ones_n, g_sp, cnt_sp, sem):
    cid = lax.axis_index("c")
    sid = lax.axis_index("s")
    w = cid * NS + sid

    io = lax.iota(i32, L)
    oh0 = jnp.where(io == 0, 1.0, 0.0).astype(f32)
    oh1 = jnp.where(io == 1, 1.0, 0.0).astype(f32)

    def init_row(r, _):
        ones_s[r, :] = oh0
        ones_n[r, :] = oh1
        return 0
    lax.fori_loop(0, C1, init_row, 0)

    pltpu.sync_copy(zd_hbm, g_sp.at[pl.ds(sid * NROW, NROW)])
    pltpu.sync_copy(z16_hbm, cnt_sp.at[pl.ds(sid * NROW, NROW)])
    plsc.subcore_barrier()

    def chunk(c, _):
        base = w * EPT1 + c * C1
        pltpu.sync_copy(si_hbm.at[pl.ds(base, C1)], si_v)
        pltpu.sync_copy(ni_hbm.at[pl.ds(base, C1)], ni_v)
        pltpu.async_copy(e_hbm.at[ni_v], rows_v, sem).wait()
        pltpu.sync_copy(rows_v, g_sp.at[si_v], add=True)
        pltpu.sync_copy(ones_s, cnt_sp.at[si_v], add=True)
        pltpu.sync_copy(ones_n, cnt_sp.at[ni_v], add=True)
        return 0
    lax.fori_loop(0, NCHUNK1, chunk, 0)

    plsc.subcore_barrier()
    pltpu.sync_copy(g_sp.at[pl.ds(sid * NROW, NROW)],
                    g_out.at[cid, pl.ds(sid * NROW, NROW)])
    pltpu.sync_copy(cnt_sp.at[pl.ds(sid * NROW, NROW)],
                    cnt_out.at[cid, pl.ds(sid * NROW, NROW)])


def _dot(a, b, ta=False, tb=False):
    dn = (((0 if ta else 1,), (1 if tb else 0,)), ((), ()))
    return lax.dot_general(a, b, dn, precision=lax.Precision.HIGHEST,
                           preferred_element_type=f32)


def _k2a_body(e_ref, gp_ref, cp_ref,
              gw0_ref, gb0_ref, gg0_ref, gbe0_ref,
              mw0_ref, mb0_ref, mg0_ref, mbe0_ref,
              fc_ref):
    e = e_ref[...]
    g = gp_ref[0, :N] + gp_ref[1, :N]
    cnt = cp_ref[0, :N] + cp_ref[1, :N]
    cs = cnt[:, 0:1]
    cn = cnt[:, 1:2]

    ssum = _dot(cs, e, ta=True)
    nsum = _dot(cn, e, ta=True)
    s_ss = _dot(e, cs * e, ta=True)
    s_nn = _dot(e, cn * e, ta=True)
    s_sn = _dot(e, g, ta=True)
    mf = f32(M)
    ones_row = jnp.ones((1, D), f32)

    def fold(w0, b0, g0, be0):
        wa = w0[:, :D]
        wb = w0[:, D:]
        m = (_dot(ssum, wa, tb=True) + _dot(nsum, wb, tb=True)) / mf + b0
        q = (_dot(ones_row, _dot(wa, s_ss) * wa, tb=True)
             + 2.0 * _dot(ones_row, _dot(wa, s_sn) * wb, tb=True)
             + _dot(ones_row, _dot(wb, s_nn) * wb, tb=True))
        eh2 = q / mf + 2.0 * b0 * (m - b0) + b0 * b0
        v = eh2 - m * m
        s = g0 * lax.rsqrt(v + 1e-5)
        t = be0 - m * s
        u = b0 * s + t
        return s, u

    sg, ug = fold(gw0_ref[...], gb0_ref[...], gg0_ref[...], gbe0_ref[...])
    sm_, um = fold(mw0_ref[...], mb0_ref[...], mg0_ref[...], mbe0_ref[...])
    fc_ref[0:1, :] = sg
    fc_ref[1:2, :] = ug
    fc_ref[2:3, :] = sm_
    fc_ref[3:4, :] = um
    fc_ref[4:8, :] = jnp.zeros((4, H), f32)


_k2a = pl.pallas_call(
    _k2a_body,
    out_shape=jax.ShapeDtypeStruct((8, H), f32),
)

BL = 2000


def _k2b_body(e_ref, fc_ref, ew_ref, powp_ref,
              gw0_ref, mw0_ref,
              as_ref, bn_ref):
    e = e_ref[...]
    sg = fc_ref[0:1, :]
    ug = fc_ref[1:2, :]
    sm_ = fc_ref[2:3, :]
    um = fc_ref[3:4, :]
    gwa = gw0_ref[:, :D]
    gwb = gw0_ref[:, D:]
    mwa = mw0_ref[:, :D]
    mwb = mw0_ref[:, D:]
    as_ref[:, :H] = _dot(e, gwa, tb=True) * sg
    as_ref[:, H:] = _dot(e, mwa, tb=True) * sm_
    bn_ref[:, :H] = _dot(e, gwb, tb=True) * sg + ug
    bn_ref[:, H:2 * H] = _dot(e, mwb, tb=True) * sm_ + um
    wpow = ew_ref[...] ** powp_ref[...]
    bn_ref[:, 2 * H:] = jnp.broadcast_to(wpow, (BL, BW - 2 * H))


_k2b = pl.pallas_call(
    _k2b_body,
    grid=(N // BL,),
    in_specs=[
        pl.BlockSpec((BL, D), lambda i: (i, 0)),
        pl.BlockSpec((8, H), lambda i: (0, 0)),
        pl.BlockSpec((BL, 1), lambda i: (i, 0)),
        pl.BlockSpec((1, 1), lambda i: (0, 0)),
        pl.BlockSpec((H, 2 * D), lambda i: (0, 0)),
        pl.BlockSpec((H, 2 * D), lambda i: (0, 0)),
    ],
    out_specs=(
        pl.BlockSpec((BL, 2 * H), lambda i: (i, 0)),
        pl.BlockSpec((BL, BW), lambda i: (i, 0)),
    ),
    out_shape=(
        jax.ShapeDtypeStruct((N, 2 * H), f32),
        jax.ShapeDtypeStruct((N, BW), f32),
    ),
)


@functools.partial(
    pl.kernel,
    out_type=jax.ShapeDtypeStruct((NC, NPAD, ZW), f32),
    mesh=_mesh,
    compiler_params=_sc_params,
    scratch_types=(
        pltpu.VMEM((4, C3), i32),
        pltpu.VMEM((4, C3), i32),
        pltpu.VMEM((2 * C3, 2 * H), f32),
        pltpu.VMEM((2 * C3, BW), f32),
        pltpu.VMEM((2 * C3, ZW), f32),
        pltpu.VMEM((H,), f32),
        pltpu.VMEM((L,), f32),
        pltpu.VMEM_SHARED((NPAD, ZW), f32),
        pltpu.SemaphoreType.DMA,
        pltpu.SemaphoreType.DMA,
        pltpu.SemaphoreType.DMA,
        pltpu.SemaphoreType.DMA,
        pltpu.SemaphoreType.DMA,
        pltpu.SemaphoreType.DMA,
        pltpu.SemaphoreType.DMA,
        pltpu.SemaphoreType.DMA,
    ),
)
def _k3(as_hbm, bn_hbm, wg_hbm, gb_hbm, si_hbm, ni_hbm, zz_hbm,
        z_out,
        si4, ni4, as_v, bn_v, zbuf, wg_v, gb_v, z_sp,
        ga0, ga1, gbs0, gbs1, sc0, sc1, ix0, ix1):
    cid = lax.axis_index("c")
    sid = lax.axis_index("s")
    w = cid * NS + sid
    tbase = w * EPT3

    ga = (ga0, ga1)
    gbs = (gbs0, gbs1)
    sc = (sc0, sc1)
    ix = (ix0, ix1)

    pltpu.sync_copy(wg_hbm, wg_v)
    pltpu.sync_copy(gb_hbm, gb_v)

    zero16 = jnp.zeros((L,), f32)

    def zrow(r, _):
        for kk in range(ZW // L):
            zbuf[r, pl.ds(kk * L, L)] = zero16
        return 0
    lax.fori_loop(0, 2 * C3, zrow, 0)

    pltpu.sync_copy(zz_hbm, z_sp.at[pl.ds(sid * NROW, NROW)])
    plsc.subcore_barrier()

    io = lax.iota(i32, L)
    gb16 = gb_v[...]

    def load_idx(slot, c):
        base = tbase + c * C3
        pltpu.sync_copy(si_hbm.at[pl.ds(base, C3)], si4.at[slot])
        pltpu.sync_copy(ni_hbm.at[pl.ds(base, C3)], ni4.at[slot])

    def fire_idx(slot, c, pd):
        base = tbase + c * C3
        pltpu.async_copy(si_hbm.at[pl.ds(base, C3)], si4.at[slot], ix[pd])
        pltpu.async_copy(ni_hbm.at[pl.ds(base, C3)], ni4.at[slot], ix[pd])

    def wait_idx(pd):
        pltpu.make_async_copy(si_hbm.at[pl.ds(0, C3)], si4.at[0],
                              ix[pd]).wait()
        pltpu.make_async_copy(ni_hbm.at[pl.ds(0, C3)], ni4.at[0],
                              ix[pd]).wait()

    def fire_gather(pd, slot):
        pltpu.async_copy(as_hbm.at[si4.at[slot]],
                         as_v.at[pl.ds(pd * C3, C3)], ga[pd])
        pltpu.async_copy(bn_hbm.at[ni4.at[slot]],
                         bn_v.at[pl.ds(pd * C3, C3)], gbs[pd])

    def wait_gather(pd):
        pltpu.make_async_copy(as_hbm.at[si4.at[0]],
                              as_v.at[pl.ds(pd * C3, C3)], ga[pd]).wait()
        pltpu.make_async_copy(bn_hbm.at[ni4.at[0]],
                              bn_v.at[pl.ds(pd * C3, C3)], gbs[pd]).wait()

    def fire_scatter(pd, slot):
        pltpu.async_copy(zbuf.at[pl.ds(pd * C3, C3)],
                         z_sp.at[si4.at[slot]], sc[pd], add=True)

    def wait_scatter(pd):
        pltpu.make_async_copy(zbuf.at[pl.ds(pd * C3, C3)],
                              z_sp.at[si4.at[0]], sc[pd]).wait()

    zvec_f = jnp.zeros((L,), f32)
    zvec_i = jnp.zeros((L,), i32)

    def silu(h):
        return h / (1.0 + jnp.exp(-h))

    def compute(pd):
        ro = pd * C3
        rows0 = io + ro
        rows1 = io + (ro + L)
        col_wp = jnp.full((L,), 2 * H, i32)
        wp0 = plsc.load_gather(bn_v, [rows0, col_wp])
        wp1 = plsc.load_gather(bn_v, [rows1, col_wp])

        @plsc.parallel_loop(
            0, H, 1, unroll=4,
            carry=(zvec_f, zvec_f, zvec_f, zvec_f, zvec_i))
        def gate_j(j, car):
            p0, p1, q0, q1, colv = car
            wgj = plsc.load_gather(wg_v, [colv])
            a0 = plsc.load_gather(as_v, [rows0, colv])
            b0 = plsc.load_gather(bn_v, [rows0, colv])
            s0 = silu(a0 + b0)
            a1 = plsc.load_gather(as_v, [rows1, colv])
            b1 = plsc.load_gather(bn_v, [rows1, colv])
            s1 = silu(a1 + b1)
            return (p1, p0 + s0 * wgj, q1, q0 + s1 * wgj, colv + 1)
        p0, p1, q0, q1, _ = gate_j

        w0 = wp0 * jnp.exp(p0 + p1 + gb16)
        w1 = wp1 * jnp.exp(q0 + q1 + gb16)
        colw = jnp.full((L,), H, i32)
        plsc.store_scatter(zbuf, [rows0, colw], w0)
        plsc.store_scatter(zbuf, [rows1, colw], w1)

        @plsc.parallel_loop(
            0, H, 1, unroll=4,
            carry=(zvec_i, jnp.full((L,), H, i32)))
        def msg_j(j, car):
            colz, colh = car
            a0 = plsc.load_gather(as_v, [rows0, colh])
            b0 = plsc.load_gather(bn_v, [rows0, colh])
            plsc.store_scatter(zbuf, [rows0, colz], w0 * silu(a0 + b0))
            a1 = plsc.load_gather(as_v, [rows1, colh])
            b1 = plsc.load_gather(bn_v, [rows1, colh])
            plsc.store_scatter(zbuf, [rows1, colz], w1 * silu(a1 + b1))
            return (colz + 1, colh + 1)
        del msg_j

    load_idx(0, 0)
    fire_gather(0, 0)
    fire_idx(1, 1, 1)

    def quad(t, _):
        for r in range(4):
            c = t * 4 + r
            pd = r % 2

            @pl.when(c >= 2)
            def _():
                wait_scatter(pd)

            @pl.when(c + 1 < NCHUNK3)
            def _():
                wait_idx((r + 1) % 2)
                fire_gather((r + 1) % 2, (r + 1) % 4)

            @pl.when(c + 2 < NCHUNK3)
            def _():
                fire_idx((r + 2) % 4, c + 2, r % 2)

            wait_gather(pd)
            compute(pd)
            fire_scatter(pd, r)
        return 0
    lax.fori_loop(0, NCHUNK3 // 4, quad, 0)

    wait_scatter(0)
    wait_scatter(1)

    plsc.subcore_barrier()
    pltpu.sync_copy(z_sp.at[pl.ds(sid * NROW, NROW)],
                    z_out.at[cid, pl.ds(sid * NROW, NROW)])


def _k4_body(zp_ref, e_ref, mwout_ref, mbout_ref, out_ref):
    z = zp_ref[0, :N, :H] + zp_ref[1, :N, :H]
    den = zp_ref[0, :N, H:H + 1] + zp_ref[1, :N, H:H + 1]
    head = (_dot(z, mwout_ref[...], tb=True) + den * mbout_ref[...])
    head = head / (den + 1e-10)
    out_ref[...] = head + e_ref[...]


_k4 = pl.pallas_call(
    _k4_body,
    out_shape=jax.ShapeDtypeStruct((N, D), f32),
)


def kernel(elem_weights, elem_in_fea, self_fea_idx, nbr_fea_idx,
           gate_W0, gate_b0, gate_g0, gate_be0, gate_Wout, gate_bout,
           msg_W0, msg_b0, msg_g0, msg_be0, msg_Wout, msg_bout, pow_param):
    e = elem_in_fea.astype(f32)
    si = self_fea_idx.astype(i32)
    ni = nbr_fea_idx.astype(i32)

    zd = jnp.zeros((NROW, D), f32)
    z16 = jnp.zeros((NROW, 16), f32)
    zz = jnp.zeros((NROW, ZW), f32)

    g_parts, cnt_parts = _k1(e, si, ni, zd, z16)

    fc = _k2a(
        e, g_parts, cnt_parts,
        gate_W0.astype(f32), gate_b0.reshape(1, H).astype(f32),
        gate_g0.reshape(1, H).astype(f32), gate_be0.reshape(1, H).astype(f32),
        msg_W0.astype(f32), msg_b0.reshape(1, H).astype(f32),
        msg_g0.reshape(1, H).astype(f32), msg_be0.reshape(1, H).astype(f32))

    as_tab, bn_tab = _k2b(
        e, fc, elem_weights.astype(f32),
        pow_param.reshape(1, 1).astype(f32),
        gate_W0.astype(f32), msg_W0.astype(f32))

    wgout = gate_Wout.reshape(H).astype(f32)
    gb16 = jnp.broadcast_to(gate_bout.astype(f32), (L,))

    npad_e = MPAD - M
    si_pad = jnp.concatenate([si, jnp.full((npad_e,), N, i32)])
    ni_pad = jnp.concatenate([ni, jnp.zeros((npad_e,), i32)])

    z_parts = _k3(as_tab, bn_tab, wgout, gb16, si_pad, ni_pad, zz)

    return _k4(z_parts, e, msg_Wout.astype(f32),
               msg_bout.reshape(1, D).astype(f32))


# speedup vs baseline: 4.1746x; 1.0889x over previous
import functools

import jax
import jax.numpy as jnp
from jax import lax
from jax.experimental import pallas as pl
from jax.experimental.pallas import tpu as pltpu
from jax.experimental.pallas import tpu_sc as plsc

N = 10000
M = 320000
D = 128
H = 128
NC = 2
NS = 16
L = 16
NW = NC * NS
NPAD = 10016
NROW = NPAD // NS
ZW = 136
BW = 272

C1 = 80
EPT1 = M // NW
NCHUNK1 = EPT1 // C1

C3 = 32
EPT3 = 10112
MPAD = EPT3 * NW
NCHUNK3 = EPT3 // C3

_mesh = plsc.VectorSubcoreMesh(core_axis_name="c", subcore_axis_name="s")
_sc_params = pltpu.CompilerParams(
    needs_layout_passes=False, use_tc_tiling_on_sc=False)

f32 = jnp.float32
i32 = jnp.int32


@functools.partial(
    pl.kernel,
    out_type=(
        jax.ShapeDtypeStruct((NC, NPAD, D), f32),
        jax.ShapeDtypeStruct((NC, NPAD, 16), f32),
    ),
    mesh=_mesh,
    compiler_params=_sc_params,
    scratch_types=(
        pltpu.VMEM((4, C1), i32),
        pltpu.VMEM((4, C1), i32),
        pltpu.VMEM((2 * C1, D), f32),
        pltpu.VMEM((C1, 16), f32),
        pltpu.VMEM((C1, 16), f32),
        pltpu.VMEM_SHARED((NPAD, D), f32),
        pltpu.VMEM_SHARED((NPAD, 16), f32),
        pltpu.SemaphoreType.DMA,
        pltpu.SemaphoreType.DMA,
        pltpu.SemaphoreType.DMA,
        pltpu.SemaphoreType.DMA,
        pltpu.SemaphoreType.DMA,
        pltpu.SemaphoreType.DMA,
    ),
)
def _k1(e_hbm, si_hbm, ni_hbm, zd_hbm, z16_hbm,
        g_out, cnt_out,
        si4, ni4, rows_v, ones_s, ones_n, g_sp, cnt_sp,
        ge0, ge1, sc0, sc1, ix0, ix1):
    cid = lax.axis_index("c")
    sid = lax.axis_index("s")
    w = cid * NS + sid
    tbase = w * EPT1

    ge = (ge0, ge1)
    sc = (sc0, sc1)
    ix = (ix0, ix1)

    io = lax.iota(i32, L)
    oh0 = jnp.where(io == 0, 1.0, 0.0).astype(f32)
    oh1 = jnp.where(io == 1, 1.0, 0.0).astype(f32)

    def init_row(r, _):
        ones_s[r, :] = oh0
        ones_n[r, :] = oh1
        return 0
    lax.fori_loop(0, C1, init_row, 0)

    pltpu.sync_copy(zd_hbm, g_sp.at[pl.ds(sid * NROW, NROW)])
    pltpu.sync_copy(z16_hbm, cnt_sp.at[pl.ds(sid * NROW, NROW)])
    plsc.subcore_barrier()

    def load_idx(slot, c):
        base = tbase + c * C1
        pltpu.sync_copy(si_hbm.at[pl.ds(base, C1)], si4.at[slot])
        pltpu.sync_copy(ni_hbm.at[pl.ds(base, C1)], ni4.at[slot])

    def fire_idx(slot, c, pd):
        base = tbase + c * C1
        pltpu.async_copy(si_hbm.at[pl.ds(base, C1)], si4.at[slot], ix[pd])
        pltpu.async_copy(ni_hbm.at[pl.ds(base, C1)], ni4.at[slot], ix[pd])

    def wait_idx(pd):
        pltpu.make_async_copy(si_hbm.at[pl.ds(0, C1)], si4.at[0],
                              ix[pd]).wait()
        pltpu.make_async_copy(ni_hbm.at[pl.ds(0, C1)], ni4.at[0],
                              ix[pd]).wait()

    def fire_gather(pd, slot):
        pltpu.async_copy(e_hbm.at[ni4.at[slot]],
                         rows_v.at[pl.ds(pd * C1, C1)], ge[pd])

    def wait_gather(pd):
        pltpu.make_async_copy(e_hbm.at[ni4.at[0]],
                              rows_v.at[pl.ds(pd * C1, C1)], ge[pd]).wait()

    def fire_scatter(pd, slot):
        pltpu.async_copy(rows_v.at[pl.ds(pd * C1, C1)],
                         g_sp.at[si4.at[slot]], sc[pd], add=True)
        pltpu.async_copy(ones_s, cnt_sp.at[si4.at[slot]], sc[pd], add=True)
        pltpu.async_copy(ones_n, cnt_sp.at[ni4.at[slot]], sc[pd], add=True)

    def wait_scatter(pd):
        pltpu.make_async_copy(rows_v.at[pl.ds(pd * C1, C1)],
                              g_sp.at[si4.at[0]], sc[pd]).wait()
        pltpu.make_async_copy(ones_s, cnt_sp.at[si4.at[0]], sc[pd]).wait()
        pltpu.make_async_copy(ones_n, cnt_sp.at[ni4.at[0]], sc[pd]).wait()

    load_idx(0, 0)
    pltpu.async_copy(e_hbm.at[ni4.at[0]],
                     rows_v.at[pl.ds(0, C1)], ge0).wait()
    pltpu.sync_copy(rows_v.at[pl.ds(0, C1)], g_sp.at[si4.at[0]], add=True)
    pltpu.sync_copy(ones_s, cnt_sp.at[si4.at[0]], add=True)
    pltpu.sync_copy(ones_n, cnt_sp.at[ni4.at[0]], add=True)

    load_idx(1, 1)
    fire_gather(1, 1)
    fire_idx(2, 2, 0)

    def quad(t, _):
        for r in range(4):
            c = t * 4 + r + 1
            slot = (r + 1) % 4
            pd = (r + 1) % 2

            @pl.when(c >= 2)
            def _():
                wait_scatter((r + 2) % 2)

            @pl.when(c + 1 < NCHUNK1)
            def _():
                wait_idx((r + 2) % 2)
                fire_gather((r + 2) % 2, (r + 2) % 4)

            @pl.when(c + 2 < NCHUNK1)
            def _():
                fire_idx((r + 3) % 4, c + 2, (r + 1) % 2)

            wait_gather(pd)
            fire_scatter(pd, slot)
        return 0
    lax.fori_loop(0, (NCHUNK1 - 1) // 4, quad, 0)

    wait_scatter((NCHUNK1 - 1) % 2)

    plsc.subcore_barrier()
    pltpu.sync_copy(g_sp.at[pl.ds(sid * NROW, NROW)],
                    g_out.at[cid, pl.ds(sid * NROW, NROW)])
    pltpu.sync_copy(cnt_sp.at[pl.ds(sid * NROW, NROW)],
                    cnt_out.at[cid, pl.ds(sid * NROW, NROW)])


def _dot(a, b, ta=False, tb=False):
    dn = (((0 if ta else 1,), (1 if tb else 0,)), ((), ()))
    return lax.dot_general(a, b, dn, precision=lax.Precision.HIGHEST,
                           preferred_element_type=f32)


def _k2a_body(e_ref, gp_ref, cp_ref,
              gw0_ref, gb0_ref, gg0_ref, gbe0_ref,
              mw0_ref, mb0_ref, mg0_ref, mbe0_ref,
              fc_ref):
    e = e_ref[...]
    g = gp_ref[0, :N] + gp_ref[1, :N]
    cnt = cp_ref[0, :N] + cp_ref[1, :N]
    cs = cnt[:, 0:1]
    cn = cnt[:, 1:2]

    ssum = _dot(cs, e, ta=True)
    nsum = _dot(cn, e, ta=True)
    s_ss = _dot(e, cs * e, ta=True)
    s_nn = _dot(e, cn * e, ta=True)
    s_sn = _dot(e, g, ta=True)
    mf = f32(M)
    ones_row = jnp.ones((1, D), f32)

    def fold(w0, b0, g0, be0):
        wa = w0[:, :D]
        wb = w0[:, D:]
        m = (_dot(ssum, wa, tb=True) + _dot(nsum, wb, tb=True)) / mf + b0
        q = (_dot(ones_row, _dot(wa, s_ss) * wa, tb=True)
             + 2.0 * _dot(ones_row, _dot(wa, s_sn) * wb, tb=True)
             + _dot(ones_row, _dot(wb, s_nn) * wb, tb=True))
        eh2 = q / mf + 2.0 * b0 * (m - b0) + b0 * b0
        v = eh2 - m * m
        s = g0 * lax.rsqrt(v + 1e-5)
        t = be0 - m * s
        u = b0 * s + t
        return s, u

    sg, ug = fold(gw0_ref[...], gb0_ref[...], gg0_ref[...], gbe0_ref[...])
    sm_, um = fold(mw0_ref[...], mb0_ref[...], mg0_ref[...], mbe0_ref[...])
    fc_ref[0:1, :] = sg
    fc_ref[1:2, :] = ug
    fc_ref[2:3, :] = sm_
    fc_ref[3:4, :] = um
    fc_ref[4:8, :] = jnp.zeros((4, H), f32)


_k2a = pl.pallas_call(
    _k2a_body,
    out_shape=jax.ShapeDtypeStruct((8, H), f32),
)

BL = 2000


def _k2b_body(e_ref, fc_ref, ew_ref, powp_ref,
              gw0_ref, mw0_ref,
              as_ref, bn_ref):
    e = e_ref[...]
    sg = fc_ref[0:1, :]
    ug = fc_ref[1:2, :]
    sm_ = fc_ref[2:3, :]
    um = fc_ref[3:4, :]
    gwa = gw0_ref[:, :D]
    gwb = gw0_ref[:, D:]
    mwa = mw0_ref[:, :D]
    mwb = mw0_ref[:, D:]
    as_ref[:, :H] = _dot(e, gwa, tb=True) * sg
    as_ref[:, H:] = _dot(e, mwa, tb=True) * sm_
    bn_ref[:, :H] = _dot(e, gwb, tb=True) * sg + ug
    bn_ref[:, H:2 * H] = _dot(e, mwb, tb=True) * sm_ + um
    wpow = ew_ref[...] ** powp_ref[...]
    bn_ref[:, 2 * H:] = jnp.broadcast_to(wpow, (BL, BW - 2 * H))


_k2b = pl.pallas_call(
    _k2b_body,
    grid=(N // BL,),
    in_specs=[
        pl.BlockSpec((BL, D), lambda i: (i, 0)),
        pl.BlockSpec((8, H), lambda i: (0, 0)),
        pl.BlockSpec((BL, 1), lambda i: (i, 0)),
        pl.BlockSpec((1, 1), lambda i: (0, 0)),
        pl.BlockSpec((H, 2 * D), lambda i: (0, 0)),
        pl.BlockSpec((H, 2 * D), lambda i: (0, 0)),
    ],
    out_specs=(
        pl.BlockSpec((BL, 2 * H), lambda i: (i, 0)),
        pl.BlockSpec((BL, BW), lambda i: (i, 0)),
    ),
    out_shape=(
        jax.ShapeDtypeStruct((N, 2 * H), f32),
        jax.ShapeDtypeStruct((N, BW), f32),
    ),
)


@functools.partial(
    pl.kernel,
    out_type=jax.ShapeDtypeStruct((NC, NPAD, ZW), f32),
    mesh=_mesh,
    compiler_params=_sc_params,
    scratch_types=(
        pltpu.VMEM((4, C3), i32),
        pltpu.VMEM((4, C3), i32),
        pltpu.VMEM((2 * C3, 2 * H), f32),
        pltpu.VMEM((2 * C3, BW), f32),
        pltpu.VMEM((2 * C3, ZW), f32),
        pltpu.VMEM((H,), f32),
        pltpu.VMEM((L,), f32),
        pltpu.VMEM_SHARED((NPAD, ZW), f32),
        pltpu.SemaphoreType.DMA,
        pltpu.SemaphoreType.DMA,
        pltpu.SemaphoreType.DMA,
        pltpu.SemaphoreType.DMA,
        pltpu.SemaphoreType.DMA,
        pltpu.SemaphoreType.DMA,
        pltpu.SemaphoreType.DMA,
        pltpu.SemaphoreType.DMA,
    ),
)
def _k3(as_hbm, bn_hbm, wg_hbm, gb_hbm, si_hbm, ni_hbm, zz_hbm,
        z_out,
        si4, ni4, as_v, bn_v, zbuf, wg_v, gb_v, z_sp,
        ga0, ga1, gbs0, gbs1, sc0, sc1, ix0, ix1):
    cid = lax.axis_index("c")
    sid = lax.axis_index("s")
    w = cid * NS + sid
    tbase = w * EPT3

    ga = (ga0, ga1)
    gbs = (gbs0, gbs1)
    sc = (sc0, sc1)
    ix = (ix0, ix1)

    pltpu.sync_copy(wg_hbm, wg_v)
    pltpu.sync_copy(gb_hbm, gb_v)

    zero16 = jnp.zeros((L,), f32)

    def zrow(r, _):
        for kk in range(ZW // L):
            zbuf[r, pl.ds(kk * L, L)] = zero16
        return 0
    lax.fori_loop(0, 2 * C3, zrow, 0)

    pltpu.sync_copy(zz_hbm, z_sp.at[pl.ds(sid * NROW, NROW)])
    plsc.subcore_barrier()

    io = lax.iota(i32, L)
    gb16 = gb_v[...]

    def load_idx(slot, c):
        base = tbase + c * C3
        pltpu.sync_copy(si_hbm.at[pl.ds(base, C3)], si4.at[slot])
        pltpu.sync_copy(ni_hbm.at[pl.ds(base, C3)], ni4.at[slot])

    def fire_idx(slot, c, pd):
        base = tbase + c * C3
        pltpu.async_copy(si_hbm.at[pl.ds(base, C3)], si4.at[slot], ix[pd])
        pltpu.async_copy(ni_hbm.at[pl.ds(base, C3)], ni4.at[slot], ix[pd])

    def wait_idx(pd):
        pltpu.make_async_copy(si_hbm.at[pl.ds(0, C3)], si4.at[0],
                              ix[pd]).wait()
        pltpu.make_async_copy(ni_hbm.at[pl.ds(0, C3)], ni4.at[0],
                              ix[pd]).wait()

    def fire_gather(pd, slot):
        pltpu.async_copy(as_hbm.at[si4.at[slot]],
                         as_v.at[pl.ds(pd * C3, C3)], ga[pd])
        pltpu.async_copy(bn_hbm.at[ni4.at[slot]],
                         bn_v.at[pl.ds(pd * C3, C3)], gbs[pd])

    def wait_gather(pd):
        pltpu.make_async_copy(as_hbm.at[si4.at[0]],
                              as_v.at[pl.ds(pd * C3, C3)], ga[pd]).wait()
        pltpu.make_async_copy(bn_hbm.at[ni4.at[0]],
                              bn_v.at[pl.ds(pd * C3, C3)], gbs[pd]).wait()

    def fire_scatter(pd, slot):
        pltpu.async_copy(zbuf.at[pl.ds(pd * C3, C3)],
                         z_sp.at[si4.at[slot]], sc[pd], add=True)

    def wait_scatter(pd):
        pltpu.make_async_copy(zbuf.at[pl.ds(pd * C3, C3)],
                              z_sp.at[si4.at[0]], sc[pd]).wait()

    zvec_f = jnp.zeros((L,), f32)
    zvec_i = jnp.zeros((L,), i32)

    def silu(h):
        return h / (1.0 + jnp.exp(-h))

    def compute(pd):
        ro = pd * C3
        rows0 = io + ro
        rows1 = io + (ro + L)
        col_wp = jnp.full((L,), 2 * H, i32)
        wp0 = plsc.load_gather(bn_v, [rows0, col_wp])
        wp1 = plsc.load_gather(bn_v, [rows1, col_wp])

        @plsc.parallel_loop(
            0, H, 1, unroll=4,
            carry=(zvec_f, zvec_f, zvec_f, zvec_f, zvec_i))
        def gate_j(j, car):
            p0, p1, q0, q1, colv = car
            wgj = plsc.load_gather(wg_v, [colv])
            a0 = plsc.load_gather(as_v, [rows0, colv])
            b0 = plsc.load_gather(bn_v, [rows0, colv])
            s0 = silu(a0 + b0)
            a1 = plsc.load_gather(as_v, [rows1, colv])
            b1 = plsc.load_gather(bn_v, [rows1, colv])
            s1 = silu(a1 + b1)
            return (p1, p0 + s0 * wgj, q1, q0 + s1 * wgj, colv + 1)
        p0, p1, q0, q1, _ = gate_j

        w0 = wp0 * jnp.exp(p0 + p1 + gb16)
        w1 = wp1 * jnp.exp(q0 + q1 + gb16)
        colw = jnp.full((L,), H, i32)
        plsc.store_scatter(zbuf, [rows0, colw], w0)
        plsc.store_scatter(zbuf, [rows1, colw], w1)

        @plsc.parallel_loop(
            0, H, 1, unroll=4,
            carry=(zvec_i, jnp.full((L,), H, i32)))
        def msg_j(j, car):
            colz, colh = car
            a0 = plsc.load_gather(as_v, [rows0, colh])
            b0 = plsc.load_gather(bn_v, [rows0, colh])
            plsc.store_scatter(zbuf, [rows0, colz], w0 * silu(a0 + b0))
            a1 = plsc.load_gather(as_v, [rows1, colh])
            b1 = plsc.load_gather(bn_v, [rows1, colh])
            plsc.store_scatter(zbuf, [rows1, colz], w1 * silu(a1 + b1))
            return (colz + 1, colh + 1)
        del msg_j

    load_idx(0, 0)
    fire_gather(0, 0)
    fire_idx(1, 1, 1)

    def quad(t, _):
        for r in range(4):
            c = t * 4 + r
            pd = r % 2

            @pl.when(c >= 2)
            def _():
                wait_scatter(pd)

            @pl.when(c + 1 < NCHUNK3)
            def _():
                wait_idx((r + 1) % 2)
                fire_gather((r + 1) % 2, (r + 1) % 4)

            @pl.when(c + 2 < NCHUNK3)
            def _():
                fire_idx((r + 2) % 4, c + 2, r % 2)

            wait_gather(pd)
            compute(pd)
            fire_scatter(pd, r)
        return 0
    lax.fori_loop(0, NCHUNK3 // 4, quad, 0)

    wait_scatter(0)
    wait_scatter(1)

    plsc.subcore_barrier()
    pltpu.sync_copy(z_sp.at[pl.ds(sid * NROW, NROW)],
                    z_out.at[cid, pl.ds(sid * NROW, NROW)])


def _k4_body(zp_ref, e_ref, mwout_ref, mbout_ref, out_ref):
    z = zp_ref[0, :N, :H] + zp_ref[1, :N, :H]
    den = zp_ref[0, :N, H:H + 1] + zp_ref[1, :N, H:H + 1]
    head = (_dot(z, mwout_ref[...], tb=True) + den * mbout_ref[...])
    head = head / (den + 1e-10)
    out_ref[...] = head + e_ref[...]


_k4 = pl.pallas_call(
    _k4_body,
    out_shape=jax.ShapeDtypeStruct((N, D), f32),
)


def kernel(elem_weights, elem_in_fea, self_fea_idx, nbr_fea_idx,
           gate_W0, gate_b0, gate_g0, gate_be0, gate_Wout, gate_bout,
           msg_W0, msg_b0, msg_g0, msg_be0, msg_Wout, msg_bout, pow_param):
    e = elem_in_fea.astype(f32)
    si = self_fea_idx.astype(i32)
    ni = nbr_fea_idx.astype(i32)

    zd = jnp.zeros((NROW, D), f32)
    z16 = jnp.zeros((NROW, 16), f32)
    zz = jnp.zeros((NROW, ZW), f32)

    g_parts, cnt_parts = _k1(e, si, ni, zd, z16)

    fc = _k2a(
        e, g_parts, cnt_parts,
        gate_W0.astype(f32), gate_b0.reshape(1, H).astype(f32),
        gate_g0.reshape(1, H).astype(f32), gate_be0.reshape(1, H).astype(f32),
        msg_W0.astype(f32), msg_b0.reshape(1, H).astype(f32),
        msg_g0.reshape(1, H).astype(f32), msg_be0.reshape(1, H).astype(f32))

    as_tab, bn_tab = _k2b(
        e, fc, elem_weights.astype(f32),
        pow_param.reshape(1, 1).astype(f32),
        gate_W0.astype(f32), msg_W0.astype(f32))

    wgout = gate_Wout.reshape(H).astype(f32)
    gb16 = jnp.broadcast_to(gate_bout.astype(f32), (L,))

    npad_e = MPAD - M
    si_pad = jnp.concatenate([si, jnp.full((npad_e,), N, i32)])
    ni_pad = jnp.concatenate([ni, jnp.zeros((npad_e,), i32)])

    z_parts = _k3(as_tab, bn_tab, wgout, gb16, si_pad, ni_pad, zz)

    return _k4(z_parts, e, msg_Wout.astype(f32),
               msg_bout.reshape(1, D).astype(f32))
